# baseline (speedup 1.0000x reference)
import jax
import jax.numpy as jnp
from jax import lax
from jax.experimental import pallas as pl
from jax.experimental.pallas import tpu as pltpu
from jax.experimental.pallas import tpu_sc as plsc

_B = 4096
_D = 32
_W = 128
_NC, _NS = 2, 16
_V = 100000
_FT = 781
_PR = (_V + 3) // 4
_TPT = 49
_BPT = _B // _NS


def _transpose_tile(in_v, tac, r0, r1):
    for q in range(32):
        for a in range(4):
            col = jnp.full((16,), 4 * q + a, jnp.int32)
            tac[q, 16 * (2 * a):16 * (2 * a) + 16] = \
                plsc.load_gather(in_v, [r0, col])
            tac[q, 16 * (2 * a + 1):16 * (2 * a + 1) + 16] = \
                plsc.load_gather(in_v, [r1, col])


def _sc_body(uhi2, ihi2, mfu, mfi, mlu, mli,
             g_mfu, g_mfi, g_mlu, g_mli, s_mfu, s_mfi, s_mlu, s_mli,
             in_a, in_b, tac_a, tac_b, idx_v, dst_a, dst_b,
             sem, wsem, gsem):
    c = lax.axis_index("c")
    s = lax.axis_index("s")
    r0 = lax.iota(jnp.int32, 16)
    r1 = r0 + 16
    is_u = c == 0

    def step(k, carry):
        @pl.when(k > 0)
        def _():
            for _i in range(2):
                pltpu.make_async_copy(
                    mfu.at[:, pl.ds(0, _W)], tac_a.at[0], wsem).wait()
                pltpu.make_async_copy(
                    mfu.at[:, pl.ds(0, _W)], tac_b.at[0], wsem).wait()

        for gg in range(2):
            t = jnp.minimum(_TPT * s + 2 * k + gg, _FT - 1)
            lane = t * _W

            @pl.when(is_u)
            def _():
                ca = pltpu.async_copy(
                    mfu.at[:, pl.ds(lane, _W)], in_a.at[gg], sem)
                cb = pltpu.async_copy(
                    mlu.at[:, pl.ds(lane, _W)], in_b.at[gg], sem)
                ca.wait()
                cb.wait()

            @pl.when(jnp.logical_not(is_u))
            def _():
                ca = pltpu.async_copy(
                    mfi.at[:, pl.ds(lane, _W)], in_a.at[gg], sem)
                cb = pltpu.async_copy(
                    mli.at[:, pl.ds(lane, _W)], in_b.at[gg], sem)
                ca.wait()
                cb.wait()

            _transpose_tile(in_a.at[gg], tac_a.at[gg], r0, r1)
            _transpose_tile(in_b.at[gg], tac_b.at[gg], r0, r1)
            row = t * 32

            @pl.when(is_u)
            def _():
                pltpu.async_copy(
                    tac_a.at[gg], s_mfu.at[pl.ds(row, 32)], wsem)
                pltpu.async_copy(
                    tac_b.at[gg], s_mlu.at[pl.ds(row, 32)], wsem)

            @pl.when(jnp.logical_not(is_u))
            def _():
                pltpu.async_copy(
                    tac_a.at[gg], s_mfi.at[pl.ds(row, 32)], wsem)
                pltpu.async_copy(
                    tac_b.at[gg], s_mli.at[pl.ds(row, 32)], wsem)
        return carry

    lax.fori_loop(0, (_TPT + 1) // 2, step, 0)
    for _i in range(2):
        pltpu.make_async_copy(mfu.at[:, pl.ds(0, _W)], tac_a.at[0],
                              wsem).wait()
        pltpu.make_async_copy(mfu.at[:, pl.ds(0, _W)], tac_b.at[0],
                              wsem).wait()
    plsc.subcore_barrier()

    @pl.when(is_u)
    def _():
        pltpu.sync_copy(uhi2.at[pl.ds(2 * s, 2)], idx_v)
        c0 = pltpu.async_copy(s_mfu.at[idx_v.at[0]],
                              dst_a.at[pl.ds(0, _W)], gsem)
        c1 = pltpu.async_copy(s_mfu.at[idx_v.at[1]],
                              dst_a.at[pl.ds(_W, _W)], gsem)
        c2 = pltpu.async_copy(s_mlu.at[idx_v.at[0]],
                              dst_b.at[pl.ds(0, _W)], gsem)
        c3 = pltpu.async_copy(s_mlu.at[idx_v.at[1]],
                              dst_b.at[pl.ds(_W, _W)], gsem)
        c0.wait()
        c1.wait()
        c2.wait()
        c3.wait()
        w0 = pltpu.async_copy(dst_a, g_mfu.at[pl.ds(_BPT * s, _BPT)], gsem)
        w1 = pltpu.async_copy(dst_b, g_mlu.at[pl.ds(_BPT * s, _BPT)], gsem)
        w0.wait()
        w1.wait()

    @pl.when(jnp.logical_not(is_u))
    def _():
        pltpu.sync_copy(ihi2.at[pl.ds(2 * s, 2)], idx_v)
        c0 = pltpu.async_copy(s_mfi.at[idx_v.at[0]],
                              dst_a.at[pl.ds(0, _W)], gsem)
        c1 = pltpu.async_copy(s_mfi.at[idx_v.at[1]],
                              dst_a.at[pl.ds(_W, _W)], gsem)
        c2 = pltpu.async_copy(s_mli.at[idx_v.at[0]],
                              dst_b.at[pl.ds(0, _W)], gsem)
        c3 = pltpu.async_copy(s_mli.at[idx_v.at[1]],
                              dst_b.at[pl.ds(_W, _W)], gsem)
        c0.wait()
        c1.wait()
        c2.wait()
        c3.wait()
        w0 = pltpu.async_copy(dst_a, g_mfi.at[pl.ds(_BPT * s, _BPT)], gsem)
        w1 = pltpu.async_copy(dst_b, g_mli.at[pl.ds(_BPT * s, _BPT)], gsem)
        w0.wait()
        w1.wait()


@jax.jit
def _sc_convert_gather(uhi2, ihi2, mfu, mfi, mlu, mli):
    mesh = plsc.VectorSubcoreMesh(
        core_axis_name="c", subcore_axis_name="s",
        num_cores=_NC, num_subcores=_NS)
    f32 = jnp.float32
    return pl.kernel(
        _sc_body,
        out_type=[jax.ShapeDtypeStruct((_B, _W), f32)] * 4
        + [jax.ShapeDtypeStruct((_PR, _W), f32)] * 4,
        mesh=mesh,
        scratch_types=[
            pltpu.VMEM((2, _D, _W), f32),
            pltpu.VMEM((2, _D, _W), f32),
            pltpu.VMEM((2, _D, _W), f32),
            pltpu.VMEM((2, _D, _W), f32),
            pltpu.VMEM((2, _W), jnp.int32),
            pltpu.VMEM((_BPT, _W), f32),
            pltpu.VMEM((_BPT, _W), f32),
            pltpu.SemaphoreType.DMA,
            pltpu.SemaphoreType.DMA,
            pltpu.SemaphoreType.DMA,
        ],
        compiler_params=pltpu.CompilerParams(
            use_tc_tiling_on_sc=True, needs_layout_passes=False),
    )(uhi2, ihi2, mfu, mfi, mlu, mli)


def _tc_mlp_body(bmfu_ref, bmfi_ref, bmlu_ref, bmli_ref, ulo_ref, ilo_ref,
                 uhi_ref, ihi_ref, l_mfu_ref, l_mfi_ref, l_mlu_ref, l_mli_ref,
                 sel_ref, w1u_ref, w1i_ref, b1_ref, w2_ref, b2_ref,
                 w3_ref, b3_ref, wp_mf_ref, wp_mlp_ref, bp_ref, out_ref):
    dot = lambda a, b: jnp.dot(a, b, preferred_element_type=jnp.float32)
    lim = _FT * 32
    oh_base = lax.broadcasted_iota(jnp.int32, (_B, (_PR - _FT * 32)), 1) + lim

    def patched(big_ref, last_ref, hi):
        big = big_ref[...]
        oh = (oh_base == hi).astype(jnp.float32)
        return jnp.where(hi >= lim, dot(oh, last_ref[...]), big)

    uhi = uhi_ref[...]
    ihi = ihi_ref[...]
    bmfu = patched(bmfu_ref, l_mfu_ref, uhi)
    bmfi = patched(bmfi_ref, l_mfi_ref, ihi)
    bmlu = patched(bmlu_ref, l_mlu_ref, uhi)
    bmli = patched(bmli_ref, l_mli_ref, ihi)
    blk = lax.broadcasted_iota(jnp.int32, (_B, _W), 1) >> 5
    mu = blk == ulo_ref[...]
    mi = blk == ilo_ref[...]
    zero = jnp.zeros((), jnp.float32)
    sel = sel_ref[...]
    mf = (dot(jnp.where(mu, bmfu, zero), sel)
          * dot(jnp.where(mi, bmfi, zero), sel))
    h = jnp.maximum(
        dot(jnp.where(mu, bmlu, zero), w1u_ref[...])
        + dot(jnp.where(mi, bmli, zero), w1i_ref[...])
        + b1_ref[...][None, :], 0.0)
    h = jnp.maximum(dot(h, w2_ref[...]) + b2_ref[...][None, :], 0.0)
    h = jnp.maximum(dot(h, w3_ref[...]) + b3_ref[...][None, :], 0.0)
    logit = (jnp.sum(mf * wp_mf_ref[...][None, :], axis=1, keepdims=True)
             + jnp.sum(h * wp_mlp_ref[...][None, :], axis=1, keepdims=True)
             + bp_ref[...][None, :])
    out_ref[...] = jax.nn.sigmoid(logit)


@jax.jit
def _tc_mlp(bmfu, bmfi, bmlu, bmli, ulo, ilo, uhi, ihi,
            l_mfu, l_mfi, l_mlu, l_mli,
            sel, w1u, w1i, b1, w2, b2, w3, b3, wp_mf, wp_mlp, bp):
    return pl.pallas_call(
        _tc_mlp_body,
        out_shape=jax.ShapeDtypeStruct((_B, 1), jnp.float32),
    )(bmfu, bmfi, bmlu, bmli, ulo, ilo, uhi, ihi,
      l_mfu, l_mfi, l_mlu, l_mli,
      sel, w1u, w1i, b1, w2, b2, w3, b3, wp_mf, wp_mlp, bp)


def kernel(user_ids, item_ids, mf_user_table, mf_item_table,
           mlp_user_table, mlp_item_table, W1, b1, W2, b2, W3, b3, Wp, bp):
    uids = user_ids.astype(jnp.int32)
    iids = item_ids.astype(jnp.int32)
    uhi = uids >> 2
    ihi = iids >> 2
    ulo = (uids & 3).reshape(_B, 1)
    ilo = (iids & 3).reshape(_B, 1)
    bmfu, bmfi, bmlu, bmli = _sc_convert_gather(
        uhi.reshape(2 * _NS, _B // (2 * _NS)),
        ihi.reshape(2 * _NS, _B // (2 * _NS)),
        mf_user_table.T, mf_item_table.T,
        mlp_user_table.T, mlp_item_table.T)[:4]
    nlast = _V - _FT * _W // 4 * 4
    del nlast
    l_mfu = mf_user_table[_FT * 32 * 4:, :].reshape(-1, _W)
    l_mfi = mf_item_table[_FT * 32 * 4:, :].reshape(-1, _W)
    l_mlu = mlp_user_table[_FT * 32 * 4:, :].reshape(-1, _W)
    l_mli = mlp_item_table[_FT * 32 * 4:, :].reshape(-1, _W)
    sel = jnp.tile(jnp.eye(_D, dtype=jnp.float32), (_W // _D, 1))
    w1u = jnp.tile(W1[:_D, :], (_W // _D, 1))
    w1i = jnp.tile(W1[_D:, :], (_W // _D, 1))
    return _tc_mlp(
        bmfu, bmfi, bmlu, bmli, ulo, ilo,
        uhi.reshape(_B, 1), ihi.reshape(_B, 1),
        l_mfu, l_mfi, l_mlu, l_mli,
        sel, w1u, w1i, b1, W2, b2, W3, b3,
        Wp[:_D, 0], Wp[_D:, 0], bp)


# speedup vs baseline: 2.2220x; 2.2220x over previous
import jax
import jax.numpy as jnp
from jax import lax
from jax.experimental import pallas as pl
from jax.experimental.pallas import tpu as pltpu
from jax.experimental.pallas import tpu_sc as plsc

_B = 4096
_D = 32
_W = 128
_NC, _NS = 2, 16
_NW = _NC * _NS
_BPW = _B // _NW


def _sc_gather_body(hi2, tab, out, idx_v, buf, sem):
    wid = lax.axis_index("s") * _NC + lax.axis_index("c")
    base = wid * _BPW
    pltpu.sync_copy(hi2.at[wid], idx_v)
    pltpu.async_copy(tab.at[idx_v], buf, sem).wait()
    pltpu.sync_copy(buf, out.at[pl.ds(base, _BPW)])


@jax.jit
def _sc_gather_one(hi2, tab):
    mesh = plsc.VectorSubcoreMesh(
        core_axis_name="c", subcore_axis_name="s",
        num_cores=_NC, num_subcores=_NS)
    return pl.kernel(
        _sc_gather_body,
        out_type=jax.ShapeDtypeStruct((_B, _W), jnp.float32),
        mesh=mesh,
        scratch_types=[
            pltpu.VMEM((_BPW,), jnp.int32),
            pltpu.VMEM((_BPW, _W), jnp.float32),
            pltpu.SemaphoreType.DMA,
        ],
        compiler_params=pltpu.CompilerParams(use_tc_tiling_on_sc=True),
    )(hi2, tab)


def _tc_mlp_body(bmfu_ref, bmfi_ref, bmlu_ref, bmli_ref, ulo_ref, ilo_ref,
                 sel_ref, w1u_ref, w1i_ref, b1_ref, w2_ref, b2_ref,
                 w3_ref, b3_ref, wp_mf_ref, wp_mlp_ref, bp_ref, out_ref):
    blk = lax.broadcasted_iota(jnp.int32, (_B, _W), 1) >> 5
    mu = blk == ulo_ref[...]
    mi = blk == ilo_ref[...]
    zero = jnp.zeros((), jnp.float32)
    dot = lambda a, b: jnp.dot(a, b, preferred_element_type=jnp.float32)
    sel = sel_ref[...]
    mf = (dot(jnp.where(mu, bmfu_ref[...], zero), sel)
          * dot(jnp.where(mi, bmfi_ref[...], zero), sel))
    h = jnp.maximum(
        dot(jnp.where(mu, bmlu_ref[...], zero), w1u_ref[...])
        + dot(jnp.where(mi, bmli_ref[...], zero), w1i_ref[...])
        + b1_ref[...][None, :], 0.0)
    h = jnp.maximum(dot(h, w2_ref[...]) + b2_ref[...][None, :], 0.0)
    h = jnp.maximum(dot(h, w3_ref[...]) + b3_ref[...][None, :], 0.0)
    logit = (jnp.sum(mf * wp_mf_ref[...][None, :], axis=1, keepdims=True)
             + jnp.sum(h * wp_mlp_ref[...][None, :], axis=1, keepdims=True)
             + bp_ref[...][None, :])
    out_ref[...] = jax.nn.sigmoid(logit)


@jax.jit
def _tc_mlp(bmfu, bmfi, bmlu, bmli, ulo, ilo,
            sel, w1u, w1i, b1, w2, b2, w3, b3, wp_mf, wp_mlp, bp):
    return pl.pallas_call(
        _tc_mlp_body,
        out_shape=jax.ShapeDtypeStruct((_B, 1), jnp.float32),
    )(bmfu, bmfi, bmlu, bmli, ulo, ilo,
      sel, w1u, w1i, b1, w2, b2, w3, b3, wp_mf, wp_mlp, bp)


def kernel(user_ids, item_ids, mf_user_table, mf_item_table,
           mlp_user_table, mlp_item_table, W1, b1, W2, b2, W3, b3, Wp, bp):
    uids = user_ids.astype(jnp.int32)
    iids = item_ids.astype(jnp.int32)
    uhi = (uids >> 2).reshape(_NW, _BPW)
    ihi = (iids >> 2).reshape(_NW, _BPW)
    ulo = (uids & 3).reshape(_B, 1)
    ilo = (iids & 3).reshape(_B, 1)
    bmfu = _sc_gather_one(uhi, mf_user_table.reshape(-1, _W))
    bmfi = _sc_gather_one(ihi, mf_item_table.reshape(-1, _W))
    bmlu = _sc_gather_one(uhi, mlp_user_table.reshape(-1, _W))
    bmli = _sc_gather_one(ihi, mlp_item_table.reshape(-1, _W))
    sel = jnp.tile(jnp.eye(_D, dtype=jnp.float32), (_W // _D, 1))
    w1u = jnp.tile(W1[:_D, :], (_W // _D, 1))
    w1i = jnp.tile(W1[_D:, :], (_W // _D, 1))
    return _tc_mlp(
        bmfu, bmfi, bmlu, bmli, ulo, ilo,
        sel, w1u, w1i, b1, W2, b2, W3, b3,
        Wp[:_D, 0], Wp[_D:, 0], bp)


# speedup vs baseline: 3.6241x; 1.6310x over previous
import jax
import jax.numpy as jnp
from jax import lax
from jax.experimental import pallas as pl
from jax.experimental.pallas import tpu as pltpu
from jax.experimental.pallas import tpu_sc as plsc

_B = 4096
_D = 32
_NC, _NS = 2, 16
_NW = _NC * _NS
_TPW = _NW // 4
_FPW = _D // _TPW
_CH = _B // 128


def _sc_gather_body(uids2, iids2, mfu, mfi, mlu, mli,
                    out_mfu, out_mfi, out_mlu, out_mli,
                    ids_v, buf, sem):
    wid = lax.axis_index("s") * _NC + lax.axis_index("c")
    table = wid // _TPW
    base_c = (wid % _TPW) * _FPW
    use_items = jnp.logical_or(table == 1, table == 3)

    @pl.when(jnp.logical_not(use_items))
    def _():
        pltpu.sync_copy(uids2, ids_v)

    @pl.when(use_items)
    def _():
        pltpu.sync_copy(iids2, ids_v)

    def gather_all(tab):
        descs = []
        for f in range(_FPW):
            row = tab.at[base_c + f]
            for j in range(_CH):
                descs.append(pltpu.async_copy(
                    row.at[ids_v.at[j]], buf.at[f, pl.ds(j * 128, 128)],
                    sem))
        for d in descs:
            d.wait()

    @pl.when(table == 0)
    def _():
        gather_all(mfu)

    @pl.when(table == 1)
    def _():
        gather_all(mfi)

    @pl.when(table == 2)
    def _():
        gather_all(mlu)

    @pl.when(table == 3)
    def _():
        gather_all(mli)

    def write_all(out):
        for f in range(_FPW):
            pltpu.sync_copy(buf.at[f], out.at[base_c + f])

    @pl.when(table == 0)
    def _():
        write_all(out_mfu)

    @pl.when(table == 1)
    def _():
        write_all(out_mfi)

    @pl.when(table == 2)
    def _():
        write_all(out_mlu)

    @pl.when(table == 3)
    def _():
        write_all(out_mli)


@jax.jit
def _sc_gather(uids2, iids2, mfu, mfi, mlu, mli):
    mesh = plsc.VectorSubcoreMesh(
        core_axis_name="c", subcore_axis_name="s",
        num_cores=_NC, num_subcores=_NS)
    f32 = jnp.float32
    return pl.kernel(
        _sc_gather_body,
        out_type=[jax.ShapeDtypeStruct((_D, _B), f32)] * 4,
        mesh=mesh,
        scratch_types=[
            pltpu.VMEM((_CH, 128), jnp.int32),
            pltpu.VMEM((_FPW, _B), f32),
            pltpu.SemaphoreType.DMA,
        ],
        compiler_params=pltpu.CompilerParams(use_tc_tiling_on_sc=False),
    )(uids2, iids2, mfu, mfi, mlu, mli)


def _tc_mlp_body(gu_ref, gi_ref, mu_ref, mi_ref,
                 w1u_ref, w1i_ref, b1_ref, w2_ref, b2_ref, w3_ref, b3_ref,
                 wp_mf_ref, wp_mlp_ref, bp_ref, out_ref):
    dot = lambda a, b: jnp.dot(a, b, preferred_element_type=jnp.float32)
    mf = gu_ref[...] * gi_ref[...]
    h = jnp.maximum(
        dot(w1u_ref[...], mu_ref[...]) + dot(w1i_ref[...], mi_ref[...])
        + b1_ref[...][:, None], 0.0)
    h = jnp.maximum(dot(w2_ref[...], h) + b2_ref[...][:, None], 0.0)
    h = jnp.maximum(dot(w3_ref[...], h) + b3_ref[...][:, None], 0.0)
    logit = (jnp.sum(mf * wp_mf_ref[...][:, None], axis=0, keepdims=True)
             + jnp.sum(h * wp_mlp_ref[...][:, None], axis=0, keepdims=True)
             + bp_ref[...][:, None])
    out_ref[...] = jax.nn.sigmoid(logit)


@jax.jit
def _tc_mlp(gu, gi, mu, mi, w1u, w1i, b1, w2, b2, w3, b3, wp_mf, wp_mlp, bp):
    return pl.pallas_call(
        _tc_mlp_body,
        out_shape=jax.ShapeDtypeStruct((1, _B), jnp.float32),
    )(gu, gi, mu, mi, w1u, w1i, b1, w2, b2, w3, b3, wp_mf, wp_mlp, bp)


def kernel(user_ids, item_ids, mf_user_table, mf_item_table,
           mlp_user_table, mlp_item_table, W1, b1, W2, b2, W3, b3, Wp, bp):
    uids2 = user_ids.astype(jnp.int32).reshape(_CH, 128)
    iids2 = item_ids.astype(jnp.int32).reshape(_CH, 128)
    gu, gi, mu, mi = _sc_gather(
        uids2, iids2,
        mf_user_table.T, mf_item_table.T,
        mlp_user_table.T, mlp_item_table.T)
    out = _tc_mlp(
        gu, gi, mu, mi,
        W1[:_D, :].T, W1[_D:, :].T, b1, W2.T, b2, W3.T, b3,
        Wp[:_D, 0], Wp[_D:, 0], bp)
    return out.reshape(_B, 1)


# speedup vs baseline: 3.6827x; 1.0162x over previous
import jax
import jax.numpy as jnp
from jax import lax
from jax.experimental import pallas as pl
from jax.experimental.pallas import tpu as pltpu
from jax.experimental.pallas import tpu_sc as plsc

_B = 4096
_D = 32
_NC, _NS = 2, 16
_NW = _NC * _NS
_TPW = _NW // 4
_FPW = _D // _TPW
_CH = _B // 512


def _sc_gather_body(uids2, iids2, mfu, mfi, mlu, mli,
                    out_mfu, out_mfi, out_mlu, out_mli,
                    ids_v, buf, sem):
    wid = lax.axis_index("s") * _NC + lax.axis_index("c")
    table = wid // _TPW
    base_c = (wid % _TPW) * _FPW
    use_items = jnp.logical_or(table == 1, table == 3)

    @pl.when(jnp.logical_not(use_items))
    def _():
        pltpu.sync_copy(uids2, ids_v)

    @pl.when(use_items)
    def _():
        pltpu.sync_copy(iids2, ids_v)

    def gather_all(tab):
        descs = []
        for f in range(_FPW):
            row = tab.at[base_c + f]
            for j in range(_CH):
                descs.append(pltpu.async_copy(
                    row.at[ids_v.at[j]], buf.at[f, pl.ds(j * 512, 512)],
                    sem))
        for d in descs:
            d.wait()

    @pl.when(table == 0)
    def _():
        gather_all(mfu)

    @pl.when(table == 1)
    def _():
        gather_all(mfi)

    @pl.when(table == 2)
    def _():
        gather_all(mlu)

    @pl.when(table == 3)
    def _():
        gather_all(mli)

    def write_all(out):
        for f in range(_FPW):
            pltpu.sync_copy(buf.at[f], out.at[base_c + f])

    @pl.when(table == 0)
    def _():
        write_all(out_mfu)

    @pl.when(table == 1)
    def _():
        write_all(out_mfi)

    @pl.when(table == 2)
    def _():
        write_all(out_mlu)

    @pl.when(table == 3)
    def _():
        write_all(out_mli)


@jax.jit
def _sc_gather(uids2, iids2, mfu, mfi, mlu, mli):
    mesh = plsc.VectorSubcoreMesh(
        core_axis_name="c", subcore_axis_name="s",
        num_cores=_NC, num_subcores=_NS)
    f32 = jnp.float32
    return pl.kernel(
        _sc_gather_body,
        out_type=[jax.ShapeDtypeStruct((_D, _B), f32)] * 4,
        mesh=mesh,
        scratch_types=[
            pltpu.VMEM((_CH, 512), jnp.int32),
            pltpu.VMEM((_FPW, _B), f32),
            pltpu.SemaphoreType.DMA,
        ],
        compiler_params=pltpu.CompilerParams(use_tc_tiling_on_sc=False),
    )(uids2, iids2, mfu, mfi, mlu, mli)


def _tc_mlp_body(gu_ref, gi_ref, mu_ref, mi_ref,
                 w1u_ref, w1i_ref, b1_ref, w2_ref, b2_ref, w3_ref, b3_ref,
                 wp_mf_ref, wp_mlp_ref, bp_ref, out_ref):
    dot = lambda a, b: jnp.dot(a, b, preferred_element_type=jnp.float32)
    mf = gu_ref[...] * gi_ref[...]
    h = jnp.maximum(
        dot(w1u_ref[...], mu_ref[...]) + dot(w1i_ref[...], mi_ref[...])
        + b1_ref[...][:, None], 0.0)
    h = jnp.maximum(dot(w2_ref[...], h) + b2_ref[...][:, None], 0.0)
    h = jnp.maximum(dot(w3_ref[...], h) + b3_ref[...][:, None], 0.0)
    logit = (jnp.sum(mf * wp_mf_ref[...][:, None], axis=0, keepdims=True)
             + jnp.sum(h * wp_mlp_ref[...][:, None], axis=0, keepdims=True)
             + bp_ref[...][:, None])
    out_ref[...] = jax.nn.sigmoid(logit)


@jax.jit
def _tc_mlp(gu, gi, mu, mi, w1u, w1i, b1, w2, b2, w3, b3, wp_mf, wp_mlp, bp):
    return pl.pallas_call(
        _tc_mlp_body,
        out_shape=jax.ShapeDtypeStruct((1, _B), jnp.float32),
    )(gu, gi, mu, mi, w1u, w1i, b1, w2, b2, w3, b3, wp_mf, wp_mlp, bp)


def kernel(user_ids, item_ids, mf_user_table, mf_item_table,
           mlp_user_table, mlp_item_table, W1, b1, W2, b2, W3, b3, Wp, bp):
    uids2 = user_ids.astype(jnp.int32).reshape(_CH, 512)
    iids2 = item_ids.astype(jnp.int32).reshape(_CH, 512)
    gu, gi, mu, mi = _sc_gather(
        uids2, iids2,
        mf_user_table.T, mf_item_table.T,
        mlp_user_table.T, mlp_item_table.T)
    out = _tc_mlp(
        gu, gi, mu, mi,
        W1[:_D, :].T, W1[_D:, :].T, b1, W2.T, b2, W3.T, b3,
        Wp[:_D, 0], Wp[_D:, 0], bp)
    return out.reshape(_B, 1)
